# baseline (speedup 1.0000x reference)
import functools

import jax
import jax.numpy as jnp
from jax import lax
from jax.experimental import pallas as pl
from jax.experimental.pallas import tpu as pltpu
from jax.experimental.pallas import tpu_sc as plsc

B, S, H = 2, 2048, 1024
E = 8
TOPK = 2
I = 2048
T = B * S
N = T * TOPK
BT = 256
PR = N + E * BT
NB = PR // BT
BS = 256

NW = 32
G_ROWS = PR // NW
G_CH = 80
C_TOK = T // NW
C_CH = 16

def _ffn(x_bf, wg, wu, wd):
    nn = (((1,), (0,)), ((), ()))
    g = lax.dot_general(x_bf, wg, nn, preferred_element_type=jnp.float32)
    u = lax.dot_general(x_bf, wu, nn, preferred_element_type=jnp.float32)
    h = (g * jax.nn.sigmoid(g) * u).astype(jnp.bfloat16)
    return lax.dot_general(h, wd, nn, preferred_element_type=jnp.float32)


@functools.cache
def _sc_kernels():
    mesh = plsc.VectorSubcoreMesh(core_axis_name="c", subcore_axis_name="s")

    @functools.partial(
        pl.kernel, mesh=mesh,
        out_type=jax.ShapeDtypeStruct((PR, H), jnp.float32),
        scratch_types=[
            pltpu.VMEM((G_CH,), jnp.int32),
            pltpu.VMEM((G_CH, H), jnp.float32),
            pltpu.SemaphoreType.DMA,
        ],
    )
    def sc_gather(x_hbm, tok_hbm, out_hbm, idx_v, rows_v, sem):
        wid = lax.axis_index("s") * 2 + lax.axis_index("c")
        base0 = wid * G_ROWS
        for j in range(G_ROWS // G_CH):
            base = base0 + j * G_CH
            pltpu.sync_copy(tok_hbm.at[pl.ds(base, G_CH)], idx_v)
            pltpu.async_copy(x_hbm.at[idx_v], rows_v, sem).wait()
            pltpu.sync_copy(rows_v, out_hbm.at[pl.ds(base, G_CH)])

    @functools.partial(
        pl.kernel, mesh=mesh,
        out_type=jax.ShapeDtypeStruct((T, H), jnp.float32),
        scratch_types=[
            pltpu.VMEM((2 * C_CH,), jnp.int32),
            pltpu.VMEM((2 * C_CH, H), jnp.float32),
            pltpu.VMEM((C_CH, H), jnp.float32),
            pltpu.SemaphoreType.DMA,
        ],
    )
    def sc_combine(sh_hbm, ys_hbm, dest_hbm, out_hbm, idx_v, rows_v, acc_v, sem):
        wid = lax.axis_index("s") * 2 + lax.axis_index("c")
        t0 = wid * C_TOK

        def chunk(j, _):
            tb = t0 + j * C_CH
            pltpu.sync_copy(dest_hbm.at[pl.ds(tb * 2, 2 * C_CH)], idx_v)
            pltpu.async_copy(ys_hbm.at[idx_v], rows_v, sem).wait()
            pltpu.sync_copy(sh_hbm.at[pl.ds(tb, C_CH)], acc_v)

            def tok(t, _):
                def col(c, _):
                    o = pl.ds(c * 16, 16)
                    acc_v[t, o] = acc_v[t, o] + rows_v[2 * t, o] + rows_v[2 * t + 1, o]
                    return 0
                return lax.fori_loop(0, H // 16, col, 0)

            lax.fori_loop(0, C_CH, tok, 0)
            pltpu.sync_copy(acc_v, out_hbm.at[pl.ds(tb, C_CH)])
            return 0

        lax.fori_loop(0, C_TOK // C_CH, chunk, 0)

    return sc_gather, sc_combine


def _grouped_body(be_ref, xs_ref, wg_ref, wu_ref, wd_ref, w_ref, out_ref):
    y = _ffn(xs_ref[...].astype(jnp.bfloat16), wg_ref[0], wu_ref[0], wd_ref[0])
    out_ref[...] = y * w_ref[...]


_grouped = pl.pallas_call(
    _grouped_body,
    grid_spec=pltpu.PrefetchScalarGridSpec(
        num_scalar_prefetch=1,
        grid=(NB,),
        in_specs=[
            pl.BlockSpec((BT, H), lambda b, be: (b, 0)),
            pl.BlockSpec((1, H, I), lambda b, be: (be[b], 0, 0)),
            pl.BlockSpec((1, H, I), lambda b, be: (be[b], 0, 0)),
            pl.BlockSpec((1, I, H), lambda b, be: (be[b], 0, 0)),
            pl.BlockSpec((BT, 1), lambda b, be: (b, 0)),
        ],
        out_specs=pl.BlockSpec((BT, H), lambda b, be: (b, 0)),
    ),
    out_shape=jax.ShapeDtypeStruct((PR, H), jnp.float32),
)


def _shared_body(x_ref, wg_ref, wu_ref, wd_ref, out_ref):
    out_ref[...] = _ffn(x_ref[...].astype(jnp.bfloat16),
                        wg_ref[...], wu_ref[...], wd_ref[...])


_shared = pl.pallas_call(
    _shared_body,
    grid=(T // BS,),
    in_specs=[
        pl.BlockSpec((BS, H), lambda b: (b, 0)),
        pl.BlockSpec((H, I), lambda b: (0, 0)),
        pl.BlockSpec((H, I), lambda b: (0, 0)),
        pl.BlockSpec((I, H), lambda b: (0, 0)),
    ],
    out_specs=pl.BlockSpec((BS, H), lambda b: (b, 0)),
    out_shape=jax.ShapeDtypeStruct((T, H), jnp.float32),
)


def kernel(x, Wg_s, Wu_s, Wd_s, gate_W, Wg, Wu, Wd):
    x_flat = x.reshape(T, H)

    logits = x_flat @ gate_W.T
    probs = jax.nn.softmax(logits, axis=-1)
    w2, e2 = lax.top_k(probs, TOPK)
    w2 = w2 / (jnp.sum(w2, axis=-1, keepdims=True) + 1e-20)

    ef = e2.reshape(-1)
    wf = w2.reshape(-1)
    onehot = (ef[:, None] == jnp.arange(E, dtype=ef.dtype)[None, :]).astype(jnp.int32)
    csum = jnp.cumsum(onehot, axis=0)
    gsz = csum[-1]
    pos_in_grp = jnp.take_along_axis(csum, ef[:, None], axis=1)[:, 0] - 1
    padded = ((gsz + BT - 1) // BT) * BT
    cpad = jnp.cumsum(padded)
    dest = ((cpad - padded)[ef] + pos_in_grp).astype(jnp.int32)
    tok_sorted = jnp.zeros((PR,), jnp.int32).at[dest].set(
        jnp.arange(N, dtype=jnp.int32) // TOPK)
    w_sorted = jnp.zeros((PR, 1), jnp.float32).at[dest, 0].set(wf)
    be = jnp.minimum(
        jnp.searchsorted(cpad, jnp.arange(NB, dtype=jnp.int32) * BT, side="right"),
        E - 1).astype(jnp.int32)

    sc_gather, sc_combine = _sc_kernels()
    xs = sc_gather(x_flat, tok_sorted)

    bf = jnp.bfloat16
    ys = _grouped(be, xs, jnp.swapaxes(Wg, 1, 2).astype(bf),
                  jnp.swapaxes(Wu, 1, 2).astype(bf),
                  jnp.swapaxes(Wd, 1, 2).astype(bf), w_sorted)
    sh = _shared(x_flat, Wg_s.T.astype(bf), Wu_s.T.astype(bf), Wd_s.T.astype(bf))
    out = sc_combine(sh, ys, dest)
    return out.reshape(B, S, H)


# speedup vs baseline: 1.1464x; 1.1464x over previous
import functools

import jax
import jax.numpy as jnp
from jax import lax
from jax.experimental import pallas as pl
from jax.experimental.pallas import tpu as pltpu
from jax.experimental.pallas import tpu_sc as plsc

B, S, H = 2, 2048, 1024
E = 8
TOPK = 2
I = 2048
T = B * S
N = T * TOPK
BT = 128
PR = N + E * BT
NB = PR // BT
BS = 256

NW = 32
G_ROWS = PR // NW
G_CH = 96
C_TOK = T // NW
C_CH = 16

def _ffn(x_bf, wg, wu, wd):
    nn = (((1,), (0,)), ((), ()))
    g = lax.dot_general(x_bf, wg, nn, preferred_element_type=jnp.float32)
    u = lax.dot_general(x_bf, wu, nn, preferred_element_type=jnp.float32)
    h = (g * jax.nn.sigmoid(g) * u).astype(jnp.bfloat16)
    return lax.dot_general(h, wd, nn, preferred_element_type=jnp.float32)


@functools.cache
def _sc_kernels():
    mesh = plsc.VectorSubcoreMesh(core_axis_name="c", subcore_axis_name="s")

    @functools.partial(
        pl.kernel, mesh=mesh,
        out_type=jax.ShapeDtypeStruct((PR, H), jnp.float32),
        scratch_types=[
            pltpu.VMEM((G_CH,), jnp.int32),
            pltpu.VMEM((G_CH, H), jnp.float32),
            pltpu.SemaphoreType.DMA,
        ],
    )
    def sc_gather(x_hbm, tok_hbm, out_hbm, idx_v, rows_v, sem):
        wid = lax.axis_index("s") * 2 + lax.axis_index("c")
        base0 = wid * G_ROWS
        for j in range(G_ROWS // G_CH):
            base = base0 + j * G_CH
            pltpu.sync_copy(tok_hbm.at[pl.ds(base, G_CH)], idx_v)
            pltpu.async_copy(x_hbm.at[idx_v], rows_v, sem).wait()
            pltpu.sync_copy(rows_v, out_hbm.at[pl.ds(base, G_CH)])

    @functools.partial(
        pl.kernel, mesh=mesh,
        out_type=jax.ShapeDtypeStruct((T, H), jnp.float32),
        scratch_types=[
            pltpu.VMEM((2 * C_CH,), jnp.int32),
            pltpu.VMEM((2 * C_CH, H), jnp.float32),
            pltpu.VMEM((C_CH, H), jnp.float32),
            pltpu.SemaphoreType.DMA,
        ],
    )
    def sc_combine(sh_hbm, ys_hbm, dest_hbm, out_hbm, idx_v, rows_v, acc_v, sem):
        wid = lax.axis_index("s") * 2 + lax.axis_index("c")
        t0 = wid * C_TOK

        def chunk(j, _):
            tb = t0 + j * C_CH
            pltpu.sync_copy(dest_hbm.at[pl.ds(tb * 2, 2 * C_CH)], idx_v)
            pltpu.async_copy(ys_hbm.at[idx_v], rows_v, sem).wait()
            pltpu.sync_copy(sh_hbm.at[pl.ds(tb, C_CH)], acc_v)

            def tok(t, _):
                def col(c, _):
                    o = pl.ds(c * 16, 16)
                    acc_v[t, o] = acc_v[t, o] + rows_v[2 * t, o] + rows_v[2 * t + 1, o]
                    return 0
                return lax.fori_loop(0, H // 16, col, 0)

            lax.fori_loop(0, C_CH, tok, 0)
            pltpu.sync_copy(acc_v, out_hbm.at[pl.ds(tb, C_CH)])
            return 0

        lax.fori_loop(0, C_TOK // C_CH, chunk, 0)

    return sc_gather, sc_combine


def _grouped_body(be_ref, xs_ref, wg_ref, wu_ref, wd_ref, w_ref, out_ref):
    y = _ffn(xs_ref[...].astype(jnp.bfloat16), wg_ref[0], wu_ref[0], wd_ref[0])
    out_ref[...] = y * w_ref[...]


_grouped = pl.pallas_call(
    _grouped_body,
    grid_spec=pltpu.PrefetchScalarGridSpec(
        num_scalar_prefetch=1,
        grid=(NB,),
        in_specs=[
            pl.BlockSpec((BT, H), lambda b, be: (b, 0)),
            pl.BlockSpec((1, H, I), lambda b, be: (be[b], 0, 0)),
            pl.BlockSpec((1, H, I), lambda b, be: (be[b], 0, 0)),
            pl.BlockSpec((1, I, H), lambda b, be: (be[b], 0, 0)),
            pl.BlockSpec((BT, 1), lambda b, be: (b, 0)),
        ],
        out_specs=pl.BlockSpec((BT, H), lambda b, be: (b, 0)),
    ),
    out_shape=jax.ShapeDtypeStruct((PR, H), jnp.float32),
)


def _shared_body(x_ref, wg_ref, wu_ref, wd_ref, out_ref):
    out_ref[...] = _ffn(x_ref[...].astype(jnp.bfloat16),
                        wg_ref[...], wu_ref[...], wd_ref[...])


_shared = pl.pallas_call(
    _shared_body,
    grid=(T // BS,),
    in_specs=[
        pl.BlockSpec((BS, H), lambda b: (b, 0)),
        pl.BlockSpec((H, I), lambda b: (0, 0)),
        pl.BlockSpec((H, I), lambda b: (0, 0)),
        pl.BlockSpec((I, H), lambda b: (0, 0)),
    ],
    out_specs=pl.BlockSpec((BS, H), lambda b: (b, 0)),
    out_shape=jax.ShapeDtypeStruct((T, H), jnp.float32),
)


def kernel(x, Wg_s, Wu_s, Wd_s, gate_W, Wg, Wu, Wd):
    x_flat = x.reshape(T, H)

    logits = x_flat @ gate_W.T
    probs = jax.nn.softmax(logits, axis=-1)
    w2, e2 = lax.top_k(probs, TOPK)
    w2 = w2 / (jnp.sum(w2, axis=-1, keepdims=True) + 1e-20)

    ef = e2.reshape(-1)
    wf = w2.reshape(-1)
    onehot = (ef[:, None] == jnp.arange(E, dtype=ef.dtype)[None, :]).astype(jnp.int32)
    csum = jnp.cumsum(onehot, axis=0)
    gsz = csum[-1]
    pos_in_grp = jnp.take_along_axis(csum, ef[:, None], axis=1)[:, 0] - 1
    padded = ((gsz + BT - 1) // BT) * BT
    cpad = jnp.cumsum(padded)
    dest = ((cpad - padded)[ef] + pos_in_grp).astype(jnp.int32)
    tokf = (jnp.arange(N, dtype=jnp.int32) // TOPK).astype(jnp.float32)
    packed = jnp.zeros((PR, 2), jnp.float32).at[dest].set(
        jnp.stack([tokf, wf], axis=1))
    tok_sorted = packed[:, 0].astype(jnp.int32)
    w_sorted = packed[:, 1:2]
    blk = jnp.arange(NB, dtype=jnp.int32) * BT
    be = jnp.minimum(jnp.sum(blk[:, None] >= cpad[None, :], axis=1),
                     E - 1).astype(jnp.int32)

    sc_gather, sc_combine = _sc_kernels()
    xs = sc_gather(x_flat, tok_sorted)

    bf = jnp.bfloat16
    ys = _grouped(be, xs, jnp.swapaxes(Wg, 1, 2).astype(bf),
                  jnp.swapaxes(Wu, 1, 2).astype(bf),
                  jnp.swapaxes(Wd, 1, 2).astype(bf), w_sorted)
    sh = _shared(x_flat, Wg_s.T.astype(bf), Wu_s.T.astype(bf), Wd_s.T.astype(bf))
    out = sc_combine(sh, ys, dest)
    return out.reshape(B, S, H)


# speedup vs baseline: 1.1622x; 1.0138x over previous
import functools

import jax
import jax.numpy as jnp
from jax import lax
from jax.experimental import pallas as pl
from jax.experimental.pallas import tpu as pltpu
from jax.experimental.pallas import tpu_sc as plsc

B, S, H = 2, 2048, 1024
E = 8
TOPK = 2
I = 2048
T = B * S
N = T * TOPK
BT = 128
PR = N + E * BT
NB = PR // BT
BS = 256

NW = 32
G_ROWS = PR // NW
G_CH = 96
C_TOK = T // NW
C_CH = 32

def _ffn(x_bf, wg, wu, wd):
    nn = (((1,), (0,)), ((), ()))
    g = lax.dot_general(x_bf, wg, nn, preferred_element_type=jnp.float32)
    u = lax.dot_general(x_bf, wu, nn, preferred_element_type=jnp.float32)
    h = (g * jax.nn.sigmoid(g) * u).astype(jnp.bfloat16)
    return lax.dot_general(h, wd, nn, preferred_element_type=jnp.float32)


@functools.cache
def _sc_kernels():
    mesh = plsc.VectorSubcoreMesh(core_axis_name="c", subcore_axis_name="s")

    @functools.partial(
        pl.kernel, mesh=mesh,
        out_type=jax.ShapeDtypeStruct((PR, H), jnp.float32),
        scratch_types=[
            pltpu.VMEM((G_CH,), jnp.int32),
            pltpu.VMEM((G_CH, H), jnp.float32),
            pltpu.SemaphoreType.DMA,
        ],
    )
    def sc_gather(x_hbm, tok_hbm, out_hbm, idx_v, rows_v, sem):
        wid = lax.axis_index("s") * 2 + lax.axis_index("c")
        base0 = wid * G_ROWS
        for j in range(G_ROWS // G_CH):
            base = base0 + j * G_CH
            pltpu.sync_copy(tok_hbm.at[pl.ds(base, G_CH)], idx_v)
            pltpu.async_copy(x_hbm.at[idx_v], rows_v, sem).wait()
            pltpu.sync_copy(rows_v, out_hbm.at[pl.ds(base, G_CH)])

    @functools.partial(
        pl.kernel, mesh=mesh,
        out_type=jax.ShapeDtypeStruct((T, H), jnp.float32),
        scratch_types=[
            pltpu.VMEM((2 * C_CH,), jnp.int32),
            pltpu.VMEM((2 * C_CH, H), jnp.float32),
            pltpu.VMEM((C_CH, H), jnp.float32),
            pltpu.SemaphoreType.DMA,
        ],
    )
    def sc_combine(sh_hbm, ys_hbm, dest_hbm, out_hbm, idx_v, rows_v, acc_v, sem):
        wid = lax.axis_index("s") * 2 + lax.axis_index("c")
        t0 = wid * C_TOK

        def chunk(j, _):
            tb = t0 + j * C_CH
            pltpu.sync_copy(dest_hbm.at[pl.ds(tb * 2, 2 * C_CH)], idx_v)
            pltpu.async_copy(ys_hbm.at[idx_v], rows_v, sem).wait()
            pltpu.sync_copy(sh_hbm.at[pl.ds(tb, C_CH)], acc_v)

            def tok(t, _):
                def col(c, _):
                    o = pl.ds(c * 16, 16)
                    acc_v[t, o] = acc_v[t, o] + rows_v[2 * t, o] + rows_v[2 * t + 1, o]
                    return 0
                return lax.fori_loop(0, H // 16, col, 0)

            lax.fori_loop(0, C_CH, tok, 0)
            pltpu.sync_copy(acc_v, out_hbm.at[pl.ds(tb, C_CH)])
            return 0

        lax.fori_loop(0, C_TOK // C_CH, chunk, 0)

    return sc_gather, sc_combine


def _grouped_body(be_ref, xs_ref, wg_ref, wu_ref, wd_ref, w_ref, out_ref):
    y = _ffn(xs_ref[...].astype(jnp.bfloat16), wg_ref[0], wu_ref[0], wd_ref[0])
    out_ref[...] = y * w_ref[...]


_grouped = pl.pallas_call(
    _grouped_body,
    grid_spec=pltpu.PrefetchScalarGridSpec(
        num_scalar_prefetch=1,
        grid=(NB,),
        in_specs=[
            pl.BlockSpec((BT, H), lambda b, be: (b, 0)),
            pl.BlockSpec((1, H, I), lambda b, be: (be[b], 0, 0)),
            pl.BlockSpec((1, H, I), lambda b, be: (be[b], 0, 0)),
            pl.BlockSpec((1, I, H), lambda b, be: (be[b], 0, 0)),
            pl.BlockSpec((BT, 1), lambda b, be: (b, 0)),
        ],
        out_specs=pl.BlockSpec((BT, H), lambda b, be: (b, 0)),
    ),
    out_shape=jax.ShapeDtypeStruct((PR, H), jnp.float32),
)


def _shared_body(x_ref, wg_ref, wu_ref, wd_ref, out_ref):
    out_ref[...] = _ffn(x_ref[...].astype(jnp.bfloat16),
                        wg_ref[...], wu_ref[...], wd_ref[...])


_shared = pl.pallas_call(
    _shared_body,
    grid=(T // BS,),
    in_specs=[
        pl.BlockSpec((BS, H), lambda b: (b, 0)),
        pl.BlockSpec((H, I), lambda b: (0, 0)),
        pl.BlockSpec((H, I), lambda b: (0, 0)),
        pl.BlockSpec((I, H), lambda b: (0, 0)),
    ],
    out_specs=pl.BlockSpec((BS, H), lambda b: (b, 0)),
    out_shape=jax.ShapeDtypeStruct((T, H), jnp.float32),
)


def kernel(x, Wg_s, Wu_s, Wd_s, gate_W, Wg, Wu, Wd):
    x_flat = x.reshape(T, H)

    logits = x_flat @ gate_W.T
    probs = jax.nn.softmax(logits, axis=-1)
    w2, e2 = lax.top_k(probs, TOPK)
    w2 = w2 / (jnp.sum(w2, axis=-1, keepdims=True) + 1e-20)

    ef = e2.reshape(-1)
    wf = w2.reshape(-1)
    onehot = (ef[:, None] == jnp.arange(E, dtype=ef.dtype)[None, :]).astype(jnp.int32)
    csum = jnp.cumsum(onehot, axis=0)
    gsz = csum[-1]
    pos_in_grp = jnp.take_along_axis(csum, ef[:, None], axis=1)[:, 0] - 1
    padded = ((gsz + BT - 1) // BT) * BT
    cpad = jnp.cumsum(padded)
    dest = ((cpad - padded)[ef] + pos_in_grp).astype(jnp.int32)
    tokf = (jnp.arange(N, dtype=jnp.int32) // TOPK).astype(jnp.float32)
    packed = jnp.zeros((PR, 2), jnp.float32).at[dest].set(
        jnp.stack([tokf, wf], axis=1))
    tok_sorted = packed[:, 0].astype(jnp.int32)
    w_sorted = packed[:, 1:2]
    blk = jnp.arange(NB, dtype=jnp.int32) * BT
    be = jnp.minimum(jnp.sum(blk[:, None] >= cpad[None, :], axis=1),
                     E - 1).astype(jnp.int32)

    sc_gather, sc_combine = _sc_kernels()
    xs = sc_gather(x_flat, tok_sorted)

    bf = jnp.bfloat16
    ys = _grouped(be, xs, jnp.swapaxes(Wg, 1, 2).astype(bf),
                  jnp.swapaxes(Wu, 1, 2).astype(bf),
                  jnp.swapaxes(Wd, 1, 2).astype(bf), w_sorted)
    sh = _shared(x_flat, Wg_s.T.astype(bf), Wu_s.T.astype(bf), Wd_s.T.astype(bf))
    out = sc_combine(sh, ys, dest)
    return out.reshape(B, S, H)
